# baseline (speedup 1.0000x reference)
import functools

import jax
import jax.numpy as jnp
from jax import lax
from jax.experimental import pallas as pl
from jax.experimental.pallas import tpu as pltpu
from jax.experimental.pallas import tpu_sc as plsc

_NC = 2
_NS = 16
_NW = _NC * _NS
_PC = 256
_SH = 8
_ICH = 2048
_K = 16


def _sweep_gather_sc(item_indices, it_t):
    B = item_indices.shape[0]
    D = it_t.shape[0]
    Vi = it_t.shape[1]
    mesh = plsc.VectorSubcoreMesh(core_axis_name="c", subcore_axis_name="s")
    np_all = (Vi + _PC - 1) // _PC
    trips = (np_all // _NW + 2) // 2

    @functools.partial(
        pl.kernel,
        mesh=mesh,
        out_type=jax.ShapeDtypeStruct((B, D), jnp.float32),
        scratch_types=[
            pltpu.VMEM((_ICH,), jnp.int32),
            pltpu.VMEM((B,), jnp.int32),
            pltpu.VMEM((B,), jnp.int32),
            pltpu.VMEM((D, _PC), jnp.float32),
            pltpu.VMEM((D, _PC), jnp.float32),
            pltpu.VMEM((32, D), jnp.float32),
            pltpu.SemaphoreType.DMA,
            pltpu.SemaphoreType.DMA,
        ],
        compiler_params=pltpu.CompilerParams(
            disable_bounds_checks=True, needs_layout_passes=False),
    )
    def gather(iidx_hbm, itab_hbm, out_hbm, ichunk, midx, mpos,
               bufa, bufb, rowbuf, psem, xsem):
        wid = lax.axis_index("s") * _NC + lax.axis_index("c")
        lanes = lax.iota(jnp.int32, 16)
        ngr = (D + 15) // 16

        def chunk(cc, off):
            pltpu.sync_copy(iidx_hbm.at[pl.ds(cc * _ICH, _ICH)], ichunk)

            def grp(g, off):
                vec = ichunk[pl.ds(g * 16, 16)]
                keym = ((vec >> _SH) & 31) == wid
                pos = lanes + (cc * _ICH + g * 16)
                pc = plsc.all_reduce_population_count(keym)
                plsc.store_compressed(midx.at[pl.ds(off, 16)], vec, mask=keym)
                plsc.store_compressed(mpos.at[pl.ds(off, 16)], pos, mask=keym)
                return off + pc[0]

            return lax.fori_loop(0, _ICH // 16, grp, off)

        m = lax.fori_loop(0, B // _ICH, chunk, 0)
        mg = (m + 15) >> 4

        def stream(p, buf):
            @pl.when(p < np_all)
            def _():
                off = pl.multiple_of(p * _PC, _PC)
                pltpu.async_copy(
                    itab_hbm.at[pl.ds(0, D), pl.ds(off, _PC)], buf, psem)

        def wait_stream(p, buf):
            @pl.when(p < np_all)
            def _():
                pltpu.make_async_copy(
                    itab_hbm.at[pl.ds(0, D), pl.ds(0, _PC)], buf, psem).wait()

        def drain_one(_, c):
            pltpu.make_async_copy(out_hbm.at[0], rowbuf.at[0], xsem).wait()
            return c

        def process(p, buf, rb):
            valid_p = p < np_all

            def scan(g, rb):
                mv = midx[pl.ds(g * 16, 16)]
                inp = ((mv >> _SH) == p) & ((lanes + g * 16) < m) & valid_p
                pc = plsc.all_reduce_population_count(inp)[0]

                @pl.when(pc > 0)
                def _():
                    pv = mpos[pl.ds(g * 16, 16)]
                    icum = plsc.cumsum(inp.astype(jnp.int32))
                    for j in range(16):
                        cond = ((mv[j] >> _SH) == p) & ((g * 16 + j) < m) & valid_p

                        @pl.when(cond)
                        def _():
                            col = mv[j] & (_PC - 1)
                            r = rb + icum[j] - 1
                            slot = r & 31

                            @pl.when(r >= 31)
                            def _():
                                drain_one(0, 0)

                            cvec = lanes * 0 + col
                            for k in range(ngr):
                                base = min(k * 16, D - 16)
                                rvec = lanes + base
                                vals = plsc.load_gather(buf, [rvec, cvec])
                                rowbuf[slot, pl.ds(base, 16)] = vals
                            pltpu.async_copy(
                                rowbuf.at[slot], out_hbm.at[pv[j]], xsem)

                return rb + pc

            return lax.fori_loop(0, mg, scan, rb)

        stream(wid, bufa)

        def trip(t, rb):
            pa = wid + 64 * t
            pb = pa + 32
            stream(pb, bufb)
            wait_stream(pa, bufa)
            rb = process(pa, bufa, rb)
            stream(pa + 64, bufa)
            wait_stream(pb, bufb)
            rb = process(pb, bufb, rb)
            return rb

        lax.fori_loop(0, trips, trip, 0)
        lax.fori_loop(0, jnp.minimum(m, 31), drain_one, 0)

    return gather(item_indices, it_t)


def _row_gather_sc(user_indices, user_table):
    B = user_indices.shape[0]
    D = user_table.shape[1]
    b_per_w = B // _NW
    half = b_per_w // 2
    mesh = plsc.VectorSubcoreMesh(core_axis_name="c", subcore_axis_name="s")

    @functools.partial(
        pl.kernel,
        mesh=mesh,
        out_type=jax.ShapeDtypeStruct((B, D), jnp.float32),
        scratch_types=[
            pltpu.VMEM((b_per_w,), jnp.int32),
            pltpu.VMEM((half, D), jnp.float32),
            pltpu.SemaphoreType.DMA,
        ],
    )
    def gather(uidx_hbm, utab_hbm, uout_hbm, uidx_v, ubuf, usem):
        wid = lax.axis_index("s") * _NC + lax.axis_index("c")
        base = wid * b_per_w
        pltpu.sync_copy(uidx_hbm.at[pl.ds(base, b_per_w)], uidx_v)

        for r in range(2):
            def issue(c, carry):
                uvec = uidx_v[pl.ds(r * half + c * _K, _K)]
                for j in range(_K):
                    pltpu.async_copy(
                        utab_hbm.at[uvec[j]], ubuf.at[c * _K + j], usem)
                return carry

            lax.fori_loop(0, half // _K, issue, 0)
            pltpu.make_async_copy(utab_hbm.at[pl.ds(0, half)], ubuf, usem).wait()
            pltpu.sync_copy(ubuf, uout_hbm.at[pl.ds(base + r * half, half)])

    return gather(user_indices, user_table)


def _mlp_body(ue_ref, ie_ref, w1u_ref, w1i_ref, b1_ref, w2_ref, b2_ref, out_ref):
    h = jnp.dot(ue_ref[...], w1u_ref[...], preferred_element_type=jnp.float32)
    h = h + jnp.dot(ie_ref[...], w1i_ref[...], preferred_element_type=jnp.float32)
    h = jnp.maximum(h + b1_ref[...], 0.0)
    out_ref[...] = jnp.dot(h, w2_ref[...], preferred_element_type=jnp.float32) + b2_ref[...]


def _mlp_tc(ue, ie, W1u, W1i, b1, W2, b2, block_b=2048):
    B, D = ue.shape
    H = W1u.shape[1]
    grid = (B // block_b,)
    return pl.pallas_call(
        _mlp_body,
        grid=grid,
        in_specs=[
            pl.BlockSpec((block_b, D), lambda i: (i, 0)),
            pl.BlockSpec((block_b, D), lambda i: (i, 0)),
            pl.BlockSpec((D, H), lambda i: (0, 0)),
            pl.BlockSpec((D, H), lambda i: (0, 0)),
            pl.BlockSpec((1, H), lambda i: (0, 0)),
            pl.BlockSpec((H, 1), lambda i: (0, 0)),
            pl.BlockSpec((1, 1), lambda i: (0, 0)),
        ],
        out_specs=pl.BlockSpec((block_b, 1), lambda i: (i, 0)),
        out_shape=jax.ShapeDtypeStruct((B, 1), jnp.float32),
    )(ue, ie, W1u, W1i, b1, W2, b2)


def kernel(user_indices, item_indices, user_table, item_table, W1, b1, W2, b2):
    D = user_table.shape[1]
    ie = _sweep_gather_sc(item_indices.astype(jnp.int32), item_table.T)
    ue = _row_gather_sc(user_indices.astype(jnp.int32), user_table)
    return _mlp_tc(
        ue, ie,
        W1[:D], W1[D:],
        b1.reshape(1, -1), W2, b2.reshape(1, 1),
    )


# speedup vs baseline: 1.4211x; 1.4211x over previous
import functools

import jax
import jax.numpy as jnp
from jax import lax
from jax.experimental import pallas as pl
from jax.experimental.pallas import tpu as pltpu
from jax.experimental.pallas import tpu_sc as plsc

_NC = 2
_NS = 16
_NW = _NC * _NS
_PC = 256
_SH = 8
_ICH = 2048
_K = 16


def _sweep_gather_sc(item_indices, it_t):
    B = item_indices.shape[0]
    D = it_t.shape[0]
    Vi = it_t.shape[1]
    mesh = plsc.VectorSubcoreMesh(core_axis_name="c", subcore_axis_name="s")
    np_all = (Vi + _PC - 1) // _PC
    trips = (np_all // _NW + 2) // 2

    @functools.partial(
        pl.kernel,
        mesh=mesh,
        out_type=jax.ShapeDtypeStruct((B, D), jnp.float32),
        scratch_types=[
            pltpu.VMEM((_ICH,), jnp.int32),
            pltpu.VMEM((B,), jnp.int32),
            pltpu.VMEM((B,), jnp.int32),
            pltpu.VMEM((D, _PC), jnp.float32),
            pltpu.VMEM((D, _PC), jnp.float32),
            pltpu.VMEM((32, D), jnp.float32),
            pltpu.SemaphoreType.DMA,
            pltpu.SemaphoreType.DMA,
        ],
        compiler_params=pltpu.CompilerParams(
            disable_bounds_checks=True, needs_layout_passes=False),
    )
    def gather(iidx_hbm, itab_hbm, out_hbm, ichunk, midx, mpos,
               bufa, bufb, rowbuf, psem, xsem):
        wid = lax.axis_index("s") * _NC + lax.axis_index("c")
        lanes = lax.iota(jnp.int32, 16)
        ngr = (D + 15) // 16

        def chunk(cc, off):
            pltpu.sync_copy(iidx_hbm.at[pl.ds(cc * _ICH, _ICH)], ichunk)

            def grp(g, off):
                vec = ichunk[pl.ds(g * 16, 16)]
                keym = ((vec >> _SH) & 31) == wid
                pos = lanes + (cc * _ICH + g * 16)
                pc = plsc.all_reduce_population_count(keym)
                plsc.store_compressed(midx.at[pl.ds(off, 16)], vec, mask=keym)
                plsc.store_compressed(mpos.at[pl.ds(off, 16)], pos, mask=keym)
                return off + pc[0]

            return lax.fori_loop(0, _ICH // 16, grp, off)

        m = lax.fori_loop(0, B // _ICH, chunk, 0)
        mg = (m + 15) >> 4

        def stream(p, buf):
            @pl.when(p < np_all)
            def _():
                off = pl.multiple_of(p * _PC, _PC)
                pltpu.async_copy(
                    itab_hbm.at[pl.ds(0, D), pl.ds(off, _PC)], buf, psem)

        def wait_stream(p, buf):
            @pl.when(p < np_all)
            def _():
                pltpu.make_async_copy(
                    itab_hbm.at[pl.ds(0, D), pl.ds(0, _PC)], buf, psem).wait()

        def drain_one(_, c):
            pltpu.make_async_copy(out_hbm.at[0], rowbuf.at[0], xsem).wait()
            return c

        def process(p, buf, rb):
            valid_p = p < np_all

            def scan(g, rb):
                mv = midx[pl.ds(g * 16, 16)]
                inp = ((mv >> _SH) == p) & ((lanes + g * 16) < m) & valid_p
                pc = plsc.all_reduce_population_count(inp)[0]

                @pl.when(pc > 0)
                def _():
                    pv = mpos[pl.ds(g * 16, 16)]
                    icum = plsc.cumsum(inp.astype(jnp.int32))
                    for j in range(16):
                        cond = ((mv[j] >> _SH) == p) & ((g * 16 + j) < m) & valid_p

                        @pl.when(cond)
                        def _():
                            col = mv[j] & (_PC - 1)
                            r = rb + icum[j] - 1
                            slot = r & 31

                            @pl.when(r >= 31)
                            def _():
                                drain_one(0, 0)

                            cvec = lanes * 0 + col
                            for k in range(ngr):
                                base = min(k * 16, D - 16)
                                rvec = lanes + base
                                vals = plsc.load_gather(buf, [rvec, cvec])
                                rowbuf[slot, pl.ds(base, 16)] = vals
                            pltpu.async_copy(
                                rowbuf.at[slot], out_hbm.at[pv[j]], xsem)

                return rb + pc

            return lax.fori_loop(0, mg, scan, rb)

        stream(wid, bufa)

        def trip(t, rb):
            pa = wid + 64 * t
            pb = pa + 32
            stream(pb, bufb)
            wait_stream(pa, bufa)
            rb = process(pa, bufa, rb)
            stream(pa + 64, bufa)
            wait_stream(pb, bufb)
            rb = process(pb, bufb, rb)
            return rb

        lax.fori_loop(0, trips, trip, 0)
        lax.fori_loop(0, jnp.minimum(m, 31), drain_one, 0)

    return gather(item_indices, it_t)


def _row_gather_sc(user_indices, user_table, order_dep):
    B = user_indices.shape[0]
    D = user_table.shape[1]
    b_per_w = B // _NW
    half = b_per_w // 2
    mesh = plsc.VectorSubcoreMesh(core_axis_name="c", subcore_axis_name="s")

    @functools.partial(
        pl.kernel,
        mesh=mesh,
        out_type=jax.ShapeDtypeStruct((B, D), jnp.float32),
        scratch_types=[
            pltpu.VMEM((b_per_w,), jnp.int32),
            pltpu.VMEM((half, D), jnp.float32),
            pltpu.SemaphoreType.DMA,
        ],
    )
    def gather(uidx_hbm, utab_hbm, dep_hbm, uout_hbm, uidx_v, ubuf, usem):
        wid = lax.axis_index("s") * _NC + lax.axis_index("c")
        base = wid * b_per_w
        pltpu.sync_copy(uidx_hbm.at[pl.ds(base, b_per_w)], uidx_v)

        for r in range(2):
            def issue(c, carry):
                uvec = uidx_v[pl.ds(r * half + c * _K, _K)]
                for j in range(_K):
                    pltpu.async_copy(
                        utab_hbm.at[uvec[j]], ubuf.at[c * _K + j], usem)
                return carry

            lax.fori_loop(0, half // _K, issue, 0)
            pltpu.make_async_copy(utab_hbm.at[pl.ds(0, half)], ubuf, usem).wait()
            pltpu.sync_copy(ubuf, uout_hbm.at[pl.ds(base + r * half, half)])

    return gather(user_indices, user_table, order_dep)


def _mlp_body(ue_ref, ie_ref, w1u_ref, w1i_ref, b1_ref, w2_ref, b2_ref, out_ref):
    h = jnp.dot(ue_ref[...], w1u_ref[...], preferred_element_type=jnp.float32)
    h = h + jnp.dot(ie_ref[...], w1i_ref[...], preferred_element_type=jnp.float32)
    h = jnp.maximum(h + b1_ref[...], 0.0)
    out_ref[...] = jnp.dot(h, w2_ref[...], preferred_element_type=jnp.float32) + b2_ref[...]


def _mlp_tc(ue, ie, W1u, W1i, b1, W2, b2, block_b=2048):
    B, D = ue.shape
    H = W1u.shape[1]
    grid = (B // block_b,)
    return pl.pallas_call(
        _mlp_body,
        grid=grid,
        in_specs=[
            pl.BlockSpec((block_b, D), lambda i: (i, 0)),
            pl.BlockSpec((block_b, D), lambda i: (i, 0)),
            pl.BlockSpec((D, H), lambda i: (0, 0)),
            pl.BlockSpec((D, H), lambda i: (0, 0)),
            pl.BlockSpec((1, H), lambda i: (0, 0)),
            pl.BlockSpec((H, 1), lambda i: (0, 0)),
            pl.BlockSpec((1, 1), lambda i: (0, 0)),
        ],
        out_specs=pl.BlockSpec((block_b, 1), lambda i: (i, 0)),
        out_shape=jax.ShapeDtypeStruct((B, 1), jnp.float32),
    )(ue, ie, W1u, W1i, b1, W2, b2)


def kernel(user_indices, item_indices, user_table, item_table, W1, b1, W2, b2):
    D = user_table.shape[1]
    ie = _sweep_gather_sc(item_indices.astype(jnp.int32), item_table.T)
    ue = _row_gather_sc(user_indices.astype(jnp.int32), user_table, ie[:1, :1])
    return _mlp_tc(
        ue, ie,
        W1[:D], W1[D:],
        b1.reshape(1, -1), W2, b2.reshape(1, 1),
    )
